# Initial kernel scaffold; baseline (speedup 1.0000x reference)
import jax
import jax.numpy as jnp
from jax.experimental import pallas as pl


def kernel(x, target):
    raise NotImplementedError("write your pallas kernel here")


# speedup vs baseline: 2.4142x; 2.4142x over previous
import functools

import jax
import jax.numpy as jnp
from jax import lax
from jax.experimental import pallas as pl
from jax.experimental.pallas import tpu as pltpu

E_SMOOTH = 0.1
BLK = 512


def _loss_block_kernel(x_ref, t_ref, out_ref, *, n_cols):
    xb = x_ref[...]
    tb = t_ref[0, 0, :]
    m = jnp.max(xb, axis=1)
    s = jnp.sum(jnp.exp(xb - m[:, None]), axis=1)
    rs = jnp.sum(xb, axis=1)
    cols = lax.broadcasted_iota(jnp.int32, xb.shape, 1)
    xt = jnp.sum(jnp.where(cols == tb[:, None], xb, 0.0), axis=1)
    partial = jnp.sum(
        m + jnp.log(s) - (E_SMOOTH / n_cols) * rs - (1.0 - E_SMOOTH) * xt
    ).reshape(1, 1)

    @pl.when(pl.program_id(0) == 0)
    def _():
        out_ref[...] = jnp.zeros((1, 1), jnp.float32)

    out_ref[...] += partial


def kernel(x, target):
    B, C = x.shape
    target = target.astype(jnp.int32)
    n_blocks = B // BLK
    t3 = target.reshape(n_blocks, 1, BLK)

    out = pl.pallas_call(
        functools.partial(_loss_block_kernel, n_cols=C),
        grid=(n_blocks,),
        in_specs=[
            pl.BlockSpec((BLK, C), lambda i: (i, 0)),
            pl.BlockSpec((1, 1, BLK), lambda i: (i, 0, 0)),
        ],
        out_specs=pl.BlockSpec((1, 1), lambda i: (0, 0)),
        out_shape=jax.ShapeDtypeStruct((1, 1), jnp.float32),
        compiler_params=pltpu.CompilerParams(
            dimension_semantics=("arbitrary",),
        ),
    )(x, t3)
    return out[0, 0] / B


# speedup vs baseline: 2.6532x; 1.0990x over previous
import functools

import jax
import jax.numpy as jnp
from jax import lax
from jax.experimental import pallas as pl
from jax.experimental.pallas import tpu as pltpu

E_SMOOTH = 0.1
BLK = 1024


def _loss_block_kernel(x_ref, t_ref, out_ref, *, n_cols):
    xb = x_ref[...]
    tb = t_ref[0, 0, :]
    m = jnp.max(xb, axis=1)
    s = jnp.sum(jnp.exp(xb - m[:, None]), axis=1)
    rs = jnp.sum(xb, axis=1)
    cols = lax.broadcasted_iota(jnp.int32, xb.shape, 1)
    xt = jnp.sum(jnp.where(cols == tb[:, None], xb, 0.0), axis=1)
    partial = jnp.sum(
        m + jnp.log(s) - (E_SMOOTH / n_cols) * rs - (1.0 - E_SMOOTH) * xt
    ).reshape(1, 1)

    @pl.when(pl.program_id(0) == 0)
    def _():
        out_ref[...] = jnp.zeros((1, 1), jnp.float32)

    out_ref[...] += partial


def kernel(x, target):
    B, C = x.shape
    target = target.astype(jnp.int32)
    n_blocks = B // BLK
    t3 = target.reshape(n_blocks, 1, BLK)

    out = pl.pallas_call(
        functools.partial(_loss_block_kernel, n_cols=C),
        grid=(n_blocks,),
        in_specs=[
            pl.BlockSpec((BLK, C), lambda i: (i, 0)),
            pl.BlockSpec((1, 1, BLK), lambda i: (i, 0, 0)),
        ],
        out_specs=pl.BlockSpec((1, 1), lambda i: (0, 0)),
        out_shape=jax.ShapeDtypeStruct((1, 1), jnp.float32),
        compiler_params=pltpu.CompilerParams(
            dimension_semantics=("arbitrary",),
        ),
    )(x, t3)
    return out[0, 0] / B


# speedup vs baseline: 2.7504x; 1.0366x over previous
import functools

import jax
import jax.numpy as jnp
from jax import lax
from jax.experimental import pallas as pl
from jax.experimental.pallas import tpu as pltpu

E_SMOOTH = 0.1
BLK = 2048


def _loss_block_kernel(x_ref, t_ref, out_ref, *, n_cols):
    xb = x_ref[...]
    tb = t_ref[0, 0, :]
    m = jnp.max(xb, axis=1)
    s = jnp.sum(jnp.exp(xb - m[:, None]), axis=1)
    rs = jnp.sum(xb, axis=1)
    cols = lax.broadcasted_iota(jnp.int32, xb.shape, 1)
    xt = jnp.sum(jnp.where(cols == tb[:, None], xb, 0.0), axis=1)
    partial = jnp.sum(
        m + jnp.log(s) - (E_SMOOTH / n_cols) * rs - (1.0 - E_SMOOTH) * xt
    ).reshape(1, 1)

    @pl.when(pl.program_id(0) == 0)
    def _():
        out_ref[...] = jnp.zeros((1, 1), jnp.float32)

    out_ref[...] += partial


def kernel(x, target):
    B, C = x.shape
    target = target.astype(jnp.int32)
    n_blocks = B // BLK
    t3 = target.reshape(n_blocks, 1, BLK)

    out = pl.pallas_call(
        functools.partial(_loss_block_kernel, n_cols=C),
        grid=(n_blocks,),
        in_specs=[
            pl.BlockSpec((BLK, C), lambda i: (i, 0)),
            pl.BlockSpec((1, 1, BLK), lambda i: (i, 0, 0)),
        ],
        out_specs=pl.BlockSpec((1, 1), lambda i: (0, 0)),
        out_shape=jax.ShapeDtypeStruct((1, 1), jnp.float32),
        compiler_params=pltpu.CompilerParams(
            dimension_semantics=("arbitrary",),
        ),
    )(x, t3)
    return out[0, 0] / B
